# Initial kernel scaffold; baseline (speedup 1.0000x reference)
import jax
import jax.numpy as jnp
from jax.experimental import pallas as pl


def kernel(x, edge_index, edge_weight, W1, b1, W2, b2, w_pi0, b_pi0, w_pi1, b_pi1, w_v, b_v):
    raise NotImplementedError("write your pallas kernel here")


# speedup vs baseline: 4.6745x; 4.6745x over previous
import functools

import jax
import jax.numpy as jnp
from jax import lax
from jax.experimental import pallas as pl
from jax.experimental.pallas import tpu as pltpu
from jax.experimental.pallas import tpu_sc as plsc

N = 10000
E = 160000
D = 256
H = 128
NC = 2
NS = 16
EB = 128
R = 1280
RPT = R // NS
MC = RPT // 8
NPT = 624
BN = 400
NB = N // BN
DEGP = 640
NDEG = NS * DEGP

_mesh = plsc.VectorSubcoreMesh(
    core_axis_name="c", subcore_axis_name="s", num_cores=NC, num_subcores=NS)


def _deg_body(src_ref, dst_ref, ew_ref, out_ref, acc, ibuf, wbuf, zbuf, sem):
    c = lax.axis_index("c")
    s = lax.axis_index("s")

    def zloop(i, _):
        zbuf[pl.ds(i * 16, 16)] = jnp.zeros((16,), jnp.float32)
        return 0
    lax.fori_loop(0, DEGP // 16, zloop, 0)
    pltpu.sync_copy(zbuf, acc.at[pl.ds(s * DEGP, DEGP)])
    plsc.subcore_barrier()

    def chunk_body(m, _):
        base = s * RPT + m * 8

        @pl.when(c == 0)
        def _():
            pltpu.sync_copy(src_ref.at[pl.ds(base, 8)], ibuf)

        @pl.when(c == 1)
        def _():
            pltpu.sync_copy(dst_ref.at[pl.ds(base, 8)], ibuf)
        pltpu.sync_copy(ew_ref.at[pl.ds(base, 8)], wbuf)
        for j in range(8):
            pltpu.sync_copy(wbuf.at[j], acc.at[ibuf.at[j]], add=True)
        return 0
    lax.fori_loop(0, MC, chunk_body, 0)
    plsc.subcore_barrier()
    pltpu.sync_copy(acc.at[pl.ds(s * DEGP, DEGP)],
                    out_ref.at[pl.ds((c * NS + s) * DEGP, DEGP)])


@functools.partial(
    pl.kernel,
    out_type=jax.ShapeDtypeStruct((NC * NDEG,), jnp.float32),
    mesh=_mesh,
    scratch_types=[
        pltpu.MemorySpace.VMEM_SHARED((NDEG,), jnp.float32),
        pltpu.VMEM((8, EB), jnp.int32),
        pltpu.VMEM((8, EB), jnp.float32),
        pltpu.VMEM((DEGP,), jnp.float32),
        pltpu.SemaphoreType.DMA,
    ],
)
def _sc_degrees(src_ref, dst_ref, ew_ref, out_ref, acc, ibuf, wbuf, zbuf, sem):
    _deg_body(src_ref, dst_ref, ew_ref, out_ref, acc, ibuf, wbuf, zbuf, sem)


def _prop_body(t_ref, src_ref, dst_ref, ew_ref, out_ref,
               acc, sbuf, dbuf, wbuf, rows, zbuf, sem):
    c = lax.axis_index("c")
    s = lax.axis_index("s")
    c_off = c * N

    def zloop(i, _):
        for k in range(H // 16):
            zbuf[i, pl.ds(k * 16, 16)] = jnp.zeros((16,), jnp.float32)
        return 0
    lax.fori_loop(0, 16, zloop, 0)
    nz = jnp.where(s == NS - 1, 40, 39)

    def zcopy(q, _):
        @pl.when(q < nz)
        def _():
            pltpu.sync_copy(zbuf, acc.at[pl.ds(s * NPT + q * 16, 16)])
        return 0
    lax.fori_loop(0, 40, zcopy, 0)
    plsc.subcore_barrier()

    def chunk_body(m, _):
        base = s * RPT + m * 8
        pltpu.sync_copy(src_ref.at[pl.ds(base, 8)], sbuf)
        pltpu.sync_copy(dst_ref.at[pl.ds(base, 8)], dbuf)
        pltpu.sync_copy(ew_ref.at[pl.ds(base * EB, 8 * EB)],
                        wbuf.at[pl.ds(0, 8 * EB)])
        for j in range(8):
            for g in range(EB // 16):
                sbuf[j, pl.ds(g * 16, 16)] = sbuf[j, pl.ds(g * 16, 16)] + c_off
        for j in range(8):
            pltpu.async_copy(t_ref.at[sbuf.at[j]], rows, sem).wait()

            def mul_body(e, _):
                coef = wbuf[pl.ds(j * EB + e, 16)][0]
                for k in range(H // 16):
                    rows[e, pl.ds(k * 16, 16)] = rows[e, pl.ds(k * 16, 16)] * coef
                return 0
            lax.fori_loop(0, EB, mul_body, 0)
            pltpu.sync_copy(rows, acc.at[dbuf.at[j]], add=True)
        return 0
    lax.fori_loop(0, MC, chunk_body, 0)
    plsc.subcore_barrier()

    @pl.when(s < NS - 1)
    def _():
        pltpu.sync_copy(acc.at[pl.ds(s * NPT, NPT)],
                        out_ref.at[pl.ds(c_off + s * NPT, NPT)])

    @pl.when(s == NS - 1)
    def _():
        pltpu.sync_copy(acc.at[pl.ds((NS - 1) * NPT, 640)],
                        out_ref.at[pl.ds(c_off + (NS - 1) * NPT, 640)])


@functools.partial(
    pl.kernel,
    out_type=jax.ShapeDtypeStruct((NC * N, H), jnp.float32),
    mesh=_mesh,
    scratch_types=[
        pltpu.MemorySpace.VMEM_SHARED((N, H), jnp.float32),
        pltpu.VMEM((8, EB), jnp.int32),
        pltpu.VMEM((8, EB), jnp.int32),
        pltpu.VMEM((8 * EB + 16,), jnp.float32),
        pltpu.VMEM((EB, H), jnp.float32),
        pltpu.VMEM((16, H), jnp.float32),
        pltpu.SemaphoreType.DMA,
    ],
)
def _sc_propagate(t_ref, src_ref, dst_ref, ew_ref, out_ref,
                  acc, sbuf, dbuf, wbuf, rows, zbuf, sem):
    _prop_body(t_ref, src_ref, dst_ref, ew_ref, out_ref,
               acc, sbuf, dbuf, wbuf, rows, zbuf, sem)


def _norm(deg_slice):
    return lax.rsqrt(jnp.maximum(deg_slice, 1.0))


def _mm1_body(x_ref, w_ref, dego_ref, o_ref):
    ns = _norm(dego_ref[...])
    t = lax.dot_general(x_ref[...], w_ref[...], (((1,), (1,)), ((), ())),
                        preferred_element_type=jnp.float32)
    o_ref[...] = t * ns


def _mm1(x, W1, deg_out):
    return pl.pallas_call(
        _mm1_body,
        grid=(NB, NC),
        in_specs=[
            pl.BlockSpec((BN, D), lambda i, c: (i, 0)),
            pl.BlockSpec((H, D), lambda i, c: (c, 0)),
            pl.BlockSpec((BN, 1), lambda i, c: (i, 0)),
        ],
        out_specs=pl.BlockSpec((BN, H), lambda i, c: (c * NB + i, 0)),
        out_shape=jax.ShapeDtypeStruct((NC * N, H), jnp.float32),
    )(x, W1, deg_out)


def _mm2_body(a0_ref, a1_ref, w_ref, b_ref, dego_ref, degi_ref, o_ref):
    nd = _norm(degi_ref[...])
    ns = _norm(dego_ref[...])
    h = jnp.concatenate([a0_ref[...], a1_ref[...]], axis=1)
    h = jnp.maximum(h * nd + b_ref[...], 0.0)
    t = lax.dot_general(h, w_ref[...], (((1,), (1,)), ((), ())),
                        preferred_element_type=jnp.float32)
    o_ref[...] = t * ns


def _mm2(agg1, W2, b1, deg_out, deg_in):
    return pl.pallas_call(
        _mm2_body,
        grid=(NB, NC),
        in_specs=[
            pl.BlockSpec((BN, H), lambda i, c: (i, 0)),
            pl.BlockSpec((BN, H), lambda i, c: (NB + i, 0)),
            pl.BlockSpec((H, D), lambda i, c: (c, 0)),
            pl.BlockSpec((1, D), lambda i, c: (0, 0)),
            pl.BlockSpec((BN, 1), lambda i, c: (i, 0)),
            pl.BlockSpec((BN, 1), lambda i, c: (i, 0)),
        ],
        out_specs=pl.BlockSpec((BN, H), lambda i, c: (c * NB + i, 0)),
        out_shape=jax.ShapeDtypeStruct((NC * N, H), jnp.float32),
    )(agg1, agg1, W2, b1, deg_out, deg_in)


def _sn_row(w):
    return w / jnp.maximum(jnp.sqrt(jnp.sum(w * w)), 1e-12)


def _heads_body(a0_ref, a1_ref, degi_ref, b2_ref,
                wp0_ref, bp0_ref, wp1_ref, bp1_ref, wv_ref, bv_ref,
                o_ref, v_ref):
    i = pl.program_id(0)
    nd = _norm(degi_ref[...])
    h2 = jnp.concatenate([a0_ref[...], a1_ref[...]], axis=1)
    h2 = h2 * nd + b2_ref[...]
    U = jnp.concatenate(
        [_sn_row(wp0_ref[...]), _sn_row(wp1_ref[...]), _sn_row(wv_ref[...]),
         jnp.zeros((H - 3, D), jnp.float32)], axis=0)
    P = lax.dot_general(h2, U, (((1,), (1,)), ((), ())),
                        preferred_element_type=jnp.float32)
    lane = lax.broadcasted_iota(jnp.int32, (BN, H), 1)
    bvec = jnp.where(lane == 0, bp0_ref[0],
                     jnp.where(lane == 1, bp1_ref[0], 0.0))
    o_ref[...] = P + bvec
    pv = jnp.sum(jnp.where(lane == 2, P, 0.0))

    @pl.when(i == 0)
    def _():
        v_ref[0] = 0.0
    v_ref[0] = v_ref[0] + pv

    @pl.when(i == NB - 1)
    def _():
        v_ref[0] = v_ref[0] / N + bv_ref[0]


def _heads(agg2, deg_in, b2, w_pi0, b_pi0, w_pi1, b_pi1, w_v, b_v):
    full = lambda i: (0, 0)
    return pl.pallas_call(
        _heads_body,
        grid=(NB,),
        in_specs=[
            pl.BlockSpec((BN, H), lambda i: (i, 0)),
            pl.BlockSpec((BN, H), lambda i: (NB + i, 0)),
            pl.BlockSpec((BN, 1), lambda i: (i, 0)),
            pl.BlockSpec((1, D), full),
            pl.BlockSpec((1, D), full),
            pl.BlockSpec(memory_space=pltpu.MemorySpace.SMEM),
            pl.BlockSpec((1, D), full),
            pl.BlockSpec(memory_space=pltpu.MemorySpace.SMEM),
            pl.BlockSpec((1, D), full),
            pl.BlockSpec(memory_space=pltpu.MemorySpace.SMEM),
        ],
        out_specs=[
            pl.BlockSpec((BN, H), lambda i: (i, 0)),
            pl.BlockSpec(memory_space=pltpu.MemorySpace.SMEM),
        ],
        out_shape=[
            jax.ShapeDtypeStruct((N, H), jnp.float32),
            jax.ShapeDtypeStruct((1,), jnp.float32),
        ],
    )(agg2, agg2, deg_in, b2, w_pi0, b_pi0, w_pi1, b_pi1, w_v, b_v)


def kernel(x, edge_index, edge_weight, W1, b1, W2, b2,
           w_pi0, b_pi0, w_pi1, b_pi1, w_v, b_v):
    pad = R * EB - E
    fill = jnp.arange(pad, dtype=jnp.int32) % N
    src2 = jnp.concatenate([edge_index[0], fill]).reshape(R, EB)
    dst2 = jnp.concatenate([edge_index[1], fill]).reshape(R, EB)
    ew1 = jnp.concatenate([edge_weight, jnp.zeros((pad,), jnp.float32)])
    ew2 = ew1.reshape(R, EB)

    deg = _sc_degrees(src2, dst2, ew2)
    deg_out = deg[:N].reshape(N, 1)
    deg_in = deg[NDEG:NDEG + N].reshape(N, 1)

    t1 = _mm1(x, W1, deg_out)
    agg1 = _sc_propagate(t1, src2, dst2, ew1)
    t2 = _mm2(agg1, W2, b1.reshape(1, D), deg_out, deg_in)
    agg2 = _sc_propagate(t2, src2, dst2, ew1)
    ph, v = _heads(agg2, deg_in, b2.reshape(1, D),
                   w_pi0, b_pi0, w_pi1, b_pi1, w_v, b_v)
    pi = jnp.concatenate([ph[:, 0:1], ph[:, 1:2]], axis=0)
    return (pi, v.reshape(1, 1))


# speedup vs baseline: 7.0976x; 1.5184x over previous
import functools

import jax
import jax.numpy as jnp
from jax import lax
from jax.experimental import pallas as pl
from jax.experimental.pallas import tpu as pltpu
from jax.experimental.pallas import tpu_sc as plsc

N = 10000
E = 160000
D = 256
H = 128
NC = 2
NS = 16
EB = 128
R = 1280
RPT = R // NS
MC = RPT // 8
NPT = 624
BN = 400
NB = N // BN
DEGP = 640
NDEG = NS * DEGP

_mesh = plsc.VectorSubcoreMesh(
    core_axis_name="c", subcore_axis_name="s", num_cores=NC, num_subcores=NS)


def _deg_body(src_ref, dst_ref, ew_ref, out_ref, acc, ibuf, wbuf, zbuf, sem):
    c = lax.axis_index("c")
    s = lax.axis_index("s")

    def zloop(i, _):
        zbuf[pl.ds(i * 16, 16)] = jnp.zeros((16,), jnp.float32)
        return 0
    lax.fori_loop(0, DEGP // 16, zloop, 0)
    pltpu.sync_copy(zbuf, acc.at[pl.ds(s * DEGP, DEGP)])
    plsc.subcore_barrier()

    def chunk_body(m, _):
        base = s * RPT + m * 8

        @pl.when(c == 0)
        def _():
            pltpu.sync_copy(src_ref.at[pl.ds(base, 8)], ibuf)

        @pl.when(c == 1)
        def _():
            pltpu.sync_copy(dst_ref.at[pl.ds(base, 8)], ibuf)
        pltpu.sync_copy(ew_ref.at[pl.ds(base, 8)], wbuf)
        for j in range(8):
            pltpu.sync_copy(wbuf.at[j], acc.at[ibuf.at[j]], add=True)
        return 0
    lax.fori_loop(0, MC, chunk_body, 0)
    plsc.subcore_barrier()
    pltpu.sync_copy(acc.at[pl.ds(s * DEGP, DEGP)],
                    out_ref.at[pl.ds((c * NS + s) * DEGP, DEGP)])


@functools.partial(
    pl.kernel,
    out_type=jax.ShapeDtypeStruct((NC * NDEG,), jnp.float32),
    mesh=_mesh,
    scratch_types=[
        pltpu.MemorySpace.VMEM_SHARED((NDEG,), jnp.float32),
        pltpu.VMEM((8, EB), jnp.int32),
        pltpu.VMEM((8, EB), jnp.float32),
        pltpu.VMEM((DEGP,), jnp.float32),
        pltpu.SemaphoreType.DMA,
    ],
)
def _sc_degrees(src_ref, dst_ref, ew_ref, out_ref, acc, ibuf, wbuf, zbuf, sem):
    _deg_body(src_ref, dst_ref, ew_ref, out_ref, acc, ibuf, wbuf, zbuf, sem)


def _prop_body(t_ref, src_ref, dst_ref, ew_ref, out_ref,
               acc, sbuf, dbuf, wbuf, rowsa, rowsb, zbuf, sema, semb):
    c = lax.axis_index("c")
    s = lax.axis_index("s")
    c_off = c * N

    def zloop(i, _):
        for k in range(H // 16):
            zbuf[i, pl.ds(k * 16, 16)] = jnp.zeros((16,), jnp.float32)
        return 0
    lax.fori_loop(0, 16, zloop, 0)
    nz = jnp.where(s == NS - 1, 40, 39)

    def zcopy(q, _):
        @pl.when(q < nz)
        def _():
            pltpu.sync_copy(zbuf, acc.at[pl.ds(s * NPT + q * 16, 16)])
        return 0
    lax.fori_loop(0, 40, zcopy, 0)
    plsc.subcore_barrier()

    bufs = (rowsa, rowsb)
    sems = (sema, semb)

    def chunk_body(m, _):
        base = s * RPT + m * 8
        pltpu.sync_copy(src_ref.at[pl.ds(base, 8)], sbuf)
        pltpu.sync_copy(dst_ref.at[pl.ds(base, 8)], dbuf)
        pltpu.sync_copy(ew_ref.at[pl.ds(base * EB, 8 * EB)], wbuf)
        for j in range(8):
            for g in range(EB // 16):
                sbuf[j, pl.ds(g * 16, 16)] = sbuf[j, pl.ds(g * 16, 16)] + c_off
        descs = {0: pltpu.async_copy(t_ref.at[sbuf.at[0]], bufs[0], sems[0])}
        for j in range(8):
            p = j % 2
            descs[p].wait()
            if j < 7:
                q = (j + 1) % 2
                descs[q] = pltpu.async_copy(
                    t_ref.at[sbuf.at[j + 1]], bufs[q], sems[q])
            buf = bufs[p]

            def mul_body(g, _):
                cvec = wbuf[pl.ds(j * EB + g * 16, 16)]
                for l in range(16):
                    e = g * 16 + l
                    coef = cvec[l]
                    for k in range(H // 16):
                        buf[e, pl.ds(k * 16, 16)] = buf[e, pl.ds(k * 16, 16)] * coef
                return 0
            lax.fori_loop(0, EB // 16, mul_body, 0)
            pltpu.sync_copy(buf, acc.at[dbuf.at[j]], add=True)
        return 0
    lax.fori_loop(0, MC, chunk_body, 0)
    plsc.subcore_barrier()

    @pl.when(s < NS - 1)
    def _():
        pltpu.sync_copy(acc.at[pl.ds(s * NPT, NPT)],
                        out_ref.at[pl.ds(c_off + s * NPT, NPT)])

    @pl.when(s == NS - 1)
    def _():
        pltpu.sync_copy(acc.at[pl.ds((NS - 1) * NPT, 640)],
                        out_ref.at[pl.ds(c_off + (NS - 1) * NPT, 640)])


@functools.partial(
    pl.kernel,
    out_type=jax.ShapeDtypeStruct((NC * N, H), jnp.float32),
    mesh=_mesh,
    scratch_types=[
        pltpu.MemorySpace.VMEM_SHARED((N, H), jnp.float32),
        pltpu.VMEM((8, EB), jnp.int32),
        pltpu.VMEM((8, EB), jnp.int32),
        pltpu.VMEM((8 * EB,), jnp.float32),
        pltpu.VMEM((EB, H), jnp.float32),
        pltpu.VMEM((EB, H), jnp.float32),
        pltpu.VMEM((16, H), jnp.float32),
        pltpu.SemaphoreType.DMA,
        pltpu.SemaphoreType.DMA,
    ],
)
def _sc_propagate(t_ref, src_ref, dst_ref, ew_ref, out_ref,
                  acc, sbuf, dbuf, wbuf, rowsa, rowsb, zbuf, sema, semb):
    _prop_body(t_ref, src_ref, dst_ref, ew_ref, out_ref,
               acc, sbuf, dbuf, wbuf, rowsa, rowsb, zbuf, sema, semb)


def _norm(deg_slice):
    return lax.rsqrt(jnp.maximum(deg_slice, 1.0))


def _mm1_body(x_ref, w_ref, dego_ref, o_ref):
    ns = _norm(dego_ref[...])
    t = lax.dot_general(x_ref[...], w_ref[...], (((1,), (1,)), ((), ())),
                        preferred_element_type=jnp.float32)
    o_ref[...] = t * ns


def _mm1(x, W1, deg_out):
    return pl.pallas_call(
        _mm1_body,
        grid=(NB, NC),
        in_specs=[
            pl.BlockSpec((BN, D), lambda i, c: (i, 0)),
            pl.BlockSpec((H, D), lambda i, c: (c, 0)),
            pl.BlockSpec((BN, 1), lambda i, c: (i, 0)),
        ],
        out_specs=pl.BlockSpec((BN, H), lambda i, c: (c * NB + i, 0)),
        out_shape=jax.ShapeDtypeStruct((NC * N, H), jnp.float32),
    )(x, W1, deg_out)


def _mm2_body(a0_ref, a1_ref, w_ref, b_ref, dego_ref, degi_ref, o_ref):
    nd = _norm(degi_ref[...])
    ns = _norm(dego_ref[...])
    h = jnp.concatenate([a0_ref[...], a1_ref[...]], axis=1)
    h = jnp.maximum(h * nd + b_ref[...], 0.0)
    t = lax.dot_general(h, w_ref[...], (((1,), (1,)), ((), ())),
                        preferred_element_type=jnp.float32)
    o_ref[...] = t * ns


def _mm2(agg1, W2, b1, deg_out, deg_in):
    return pl.pallas_call(
        _mm2_body,
        grid=(NB, NC),
        in_specs=[
            pl.BlockSpec((BN, H), lambda i, c: (i, 0)),
            pl.BlockSpec((BN, H), lambda i, c: (NB + i, 0)),
            pl.BlockSpec((H, D), lambda i, c: (c, 0)),
            pl.BlockSpec((1, D), lambda i, c: (0, 0)),
            pl.BlockSpec((BN, 1), lambda i, c: (i, 0)),
            pl.BlockSpec((BN, 1), lambda i, c: (i, 0)),
        ],
        out_specs=pl.BlockSpec((BN, H), lambda i, c: (c * NB + i, 0)),
        out_shape=jax.ShapeDtypeStruct((NC * N, H), jnp.float32),
    )(agg1, agg1, W2, b1, deg_out, deg_in)


def _sn_row(w):
    return w / jnp.maximum(jnp.sqrt(jnp.sum(w * w)), 1e-12)


def _heads_body(a0_ref, a1_ref, degi_ref, b2_ref,
                wp0_ref, bp0_ref, wp1_ref, bp1_ref, wv_ref, bv_ref,
                o_ref, v_ref):
    i = pl.program_id(0)
    nd = _norm(degi_ref[...])
    h2 = jnp.concatenate([a0_ref[...], a1_ref[...]], axis=1)
    h2 = h2 * nd + b2_ref[...]
    U = jnp.concatenate(
        [_sn_row(wp0_ref[...]), _sn_row(wp1_ref[...]), _sn_row(wv_ref[...]),
         jnp.zeros((H - 3, D), jnp.float32)], axis=0)
    P = lax.dot_general(h2, U, (((1,), (1,)), ((), ())),
                        preferred_element_type=jnp.float32)
    lane = lax.broadcasted_iota(jnp.int32, (BN, H), 1)
    bvec = jnp.where(lane == 0, bp0_ref[0],
                     jnp.where(lane == 1, bp1_ref[0], 0.0))
    o_ref[...] = P + bvec
    pv = jnp.sum(jnp.where(lane == 2, P, 0.0))

    @pl.when(i == 0)
    def _():
        v_ref[0] = 0.0
    v_ref[0] = v_ref[0] + pv

    @pl.when(i == NB - 1)
    def _():
        v_ref[0] = v_ref[0] / N + bv_ref[0]


def _heads(agg2, deg_in, b2, w_pi0, b_pi0, w_pi1, b_pi1, w_v, b_v):
    full = lambda i: (0, 0)
    return pl.pallas_call(
        _heads_body,
        grid=(NB,),
        in_specs=[
            pl.BlockSpec((BN, H), lambda i: (i, 0)),
            pl.BlockSpec((BN, H), lambda i: (NB + i, 0)),
            pl.BlockSpec((BN, 1), lambda i: (i, 0)),
            pl.BlockSpec((1, D), full),
            pl.BlockSpec((1, D), full),
            pl.BlockSpec(memory_space=pltpu.MemorySpace.SMEM),
            pl.BlockSpec((1, D), full),
            pl.BlockSpec(memory_space=pltpu.MemorySpace.SMEM),
            pl.BlockSpec((1, D), full),
            pl.BlockSpec(memory_space=pltpu.MemorySpace.SMEM),
        ],
        out_specs=[
            pl.BlockSpec((BN, H), lambda i: (i, 0)),
            pl.BlockSpec(memory_space=pltpu.MemorySpace.SMEM),
        ],
        out_shape=[
            jax.ShapeDtypeStruct((N, H), jnp.float32),
            jax.ShapeDtypeStruct((1,), jnp.float32),
        ],
    )(agg2, agg2, deg_in, b2, w_pi0, b_pi0, w_pi1, b_pi1, w_v, b_v)


def kernel(x, edge_index, edge_weight, W1, b1, W2, b2,
           w_pi0, b_pi0, w_pi1, b_pi1, w_v, b_v):
    pad = R * EB - E
    fill = jnp.arange(pad, dtype=jnp.int32) % N
    src2 = jnp.concatenate([edge_index[0], fill]).reshape(R, EB)
    dst2 = jnp.concatenate([edge_index[1], fill]).reshape(R, EB)
    ew1 = jnp.concatenate([edge_weight, jnp.zeros((pad,), jnp.float32)])
    ew2 = ew1.reshape(R, EB)

    deg = _sc_degrees(src2, dst2, ew2)
    deg_out = deg[:N].reshape(N, 1)
    deg_in = deg[NDEG:NDEG + N].reshape(N, 1)

    t1 = _mm1(x, W1, deg_out)
    agg1 = _sc_propagate(t1, src2, dst2, ew1)
    t2 = _mm2(agg1, W2, b1.reshape(1, D), deg_out, deg_in)
    agg2 = _sc_propagate(t2, src2, dst2, ew1)
    ph, v = _heads(agg2, deg_in, b2.reshape(1, D),
                   w_pi0, b_pi0, w_pi1, b_pi1, w_v, b_v)
    pi = jnp.concatenate([ph[:, 0:1], ph[:, 1:2]], axis=0)
    return (pi, v.reshape(1, 1))


# speedup vs baseline: 7.1113x; 1.0019x over previous
import functools

import jax
import jax.numpy as jnp
from jax import lax
from jax.experimental import pallas as pl
from jax.experimental.pallas import tpu as pltpu
from jax.experimental.pallas import tpu_sc as plsc

N = 10000
E = 160000
D = 256
H = 128
NC = 2
NS = 16
EB = 128
R = 1280
RPT = R // NS
MC = RPT // 8
NPT = 624
BN = 400
NB = N // BN
DEGP = 640
NDEG = NS * DEGP

_mesh = plsc.VectorSubcoreMesh(
    core_axis_name="c", subcore_axis_name="s", num_cores=NC, num_subcores=NS)


def _deg_body(src_ref, dst_ref, ew_ref, out_ref, acc, ibuf, wbuf, zbuf, sem):
    c = lax.axis_index("c")
    s = lax.axis_index("s")

    def zloop(i, _):
        zbuf[pl.ds(i * 16, 16)] = jnp.zeros((16,), jnp.float32)
        return 0
    lax.fori_loop(0, DEGP // 16, zloop, 0)
    pltpu.sync_copy(zbuf, acc.at[pl.ds(s * DEGP, DEGP)])
    plsc.subcore_barrier()

    def chunk_body(m, _):
        base = s * RPT + m * 8

        @pl.when(c == 0)
        def _():
            pltpu.sync_copy(src_ref.at[pl.ds(base, 8)], ibuf)

        @pl.when(c == 1)
        def _():
            pltpu.sync_copy(dst_ref.at[pl.ds(base, 8)], ibuf)
        pltpu.sync_copy(ew_ref.at[pl.ds(base, 8)], wbuf)
        for j in range(8):
            pltpu.sync_copy(wbuf.at[j], acc.at[ibuf.at[j]], add=True)
        return 0
    lax.fori_loop(0, MC, chunk_body, 0)
    plsc.subcore_barrier()
    pltpu.sync_copy(acc.at[pl.ds(s * DEGP, DEGP)],
                    out_ref.at[pl.ds((c * NS + s) * DEGP, DEGP)])


@functools.partial(
    pl.kernel,
    out_type=jax.ShapeDtypeStruct((NC * NDEG,), jnp.float32),
    mesh=_mesh,
    scratch_types=[
        pltpu.MemorySpace.VMEM_SHARED((NDEG,), jnp.float32),
        pltpu.VMEM((8, EB), jnp.int32),
        pltpu.VMEM((8, EB), jnp.float32),
        pltpu.VMEM((DEGP,), jnp.float32),
        pltpu.SemaphoreType.DMA,
    ],
)
def _sc_degrees(src_ref, dst_ref, ew_ref, out_ref, acc, ibuf, wbuf, zbuf, sem):
    _deg_body(src_ref, dst_ref, ew_ref, out_ref, acc, ibuf, wbuf, zbuf, sem)


def _prop_body(t_ref, src_ref, dst_ref, ew_ref, out_ref,
               acc, sbuf, dbuf, wbuf, rowsa, rowsb, gsa, gsb, ssa, ssb):
    c = lax.axis_index("c")
    s = lax.axis_index("s")
    c_off = c * N

    def zloop(i, _):
        for k in range(H // 16):
            rowsa[i, pl.ds(k * 16, 16)] = jnp.zeros((16,), jnp.float32)
        return 0
    lax.fori_loop(0, EB, zloop, 0)
    for q in range(4):
        pltpu.sync_copy(rowsa, acc.at[pl.ds(s * NPT + q * EB, EB)])

    @pl.when(s < NS - 1)
    def _():
        pltpu.sync_copy(rowsa.at[pl.ds(0, NPT - 4 * EB)],
                        acc.at[pl.ds(s * NPT + 4 * EB, NPT - 4 * EB)])

    @pl.when(s == NS - 1)
    def _():
        pltpu.sync_copy(rowsa, acc.at[pl.ds((NS - 1) * NPT + 4 * EB, EB)])
    plsc.subcore_barrier()

    bufs = (rowsa, rowsb)
    gsems = (gsa, gsb)
    ssems = (ssa, ssb)

    def chunk_body(m, _):
        base = s * RPT + m * 8
        pltpu.sync_copy(src_ref.at[pl.ds(base, 8)], sbuf)
        pltpu.sync_copy(dst_ref.at[pl.ds(base, 8)], dbuf)
        pltpu.sync_copy(ew_ref.at[pl.ds(base * EB, 8 * EB)], wbuf)

        @pl.when(c == 1)
        def _():
            for j in range(8):
                for g in range(EB // 16):
                    sbuf[j, pl.ds(g * 16, 16)] = sbuf[j, pl.ds(g * 16, 16)] + c_off
        dg = {0: pltpu.async_copy(t_ref.at[sbuf.at[0]], rowsa, gsa)}
        dpend = {}
        for j in range(8):
            p = j & 1
            dg[j].wait()
            if j < 7:
                q = (j + 1) & 1
                if j >= 1:
                    dpend[q].wait()
                dg[j + 1] = pltpu.async_copy(
                    t_ref.at[sbuf.at[j + 1]], bufs[q], gsems[q])
            buf = bufs[p]

            def mul_body(g, _):
                cvec = wbuf[pl.ds(j * EB + g * 16, 16)]
                for l in range(16):
                    e = g * 16 + l
                    coef = cvec[l]
                    for k in range(H // 16):
                        buf[e, pl.ds(k * 16, 16)] = buf[e, pl.ds(k * 16, 16)] * coef
                return 0
            lax.fori_loop(0, EB // 16, mul_body, 0)
            dpend[p] = pltpu.async_copy(buf, acc.at[dbuf.at[j]],
                                        ssems[p], add=True)
        dpend[0].wait()
        dpend[1].wait()
        return 0
    lax.fori_loop(0, MC, chunk_body, 0)
    plsc.subcore_barrier()

    @pl.when(s < NS - 1)
    def _():
        pltpu.sync_copy(acc.at[pl.ds(s * NPT, NPT)],
                        out_ref.at[pl.ds(c_off + s * NPT, NPT)])

    @pl.when(s == NS - 1)
    def _():
        pltpu.sync_copy(acc.at[pl.ds((NS - 1) * NPT, 640)],
                        out_ref.at[pl.ds(c_off + (NS - 1) * NPT, 640)])


@functools.partial(
    pl.kernel,
    out_type=jax.ShapeDtypeStruct((NC * N, H), jnp.float32),
    mesh=_mesh,
    scratch_types=[
        pltpu.MemorySpace.VMEM_SHARED((N, H), jnp.float32),
        pltpu.VMEM((8, EB), jnp.int32),
        pltpu.VMEM((8, EB), jnp.int32),
        pltpu.VMEM((8 * EB,), jnp.float32),
        pltpu.VMEM((EB, H), jnp.float32),
        pltpu.VMEM((EB, H), jnp.float32),
        pltpu.SemaphoreType.DMA,
        pltpu.SemaphoreType.DMA,
        pltpu.SemaphoreType.DMA,
        pltpu.SemaphoreType.DMA,
    ],
)
def _sc_propagate(t_ref, src_ref, dst_ref, ew_ref, out_ref,
                  acc, sbuf, dbuf, wbuf, rowsa, rowsb, gsa, gsb, ssa, ssb):
    _prop_body(t_ref, src_ref, dst_ref, ew_ref, out_ref,
               acc, sbuf, dbuf, wbuf, rowsa, rowsb, gsa, gsb, ssa, ssb)


def _norm(deg_slice):
    return lax.rsqrt(jnp.maximum(deg_slice, 1.0))


def _mm1_body(x_ref, w_ref, dego_ref, o_ref):
    ns = _norm(dego_ref[...])
    t = lax.dot_general(x_ref[...], w_ref[...], (((1,), (1,)), ((), ())),
                        preferred_element_type=jnp.float32)
    o_ref[...] = t * ns


def _mm1(x, W1, deg_out):
    return pl.pallas_call(
        _mm1_body,
        grid=(NB, NC),
        in_specs=[
            pl.BlockSpec((BN, D), lambda i, c: (i, 0)),
            pl.BlockSpec((H, D), lambda i, c: (c, 0)),
            pl.BlockSpec((BN, 1), lambda i, c: (i, 0)),
        ],
        out_specs=pl.BlockSpec((BN, H), lambda i, c: (c * NB + i, 0)),
        out_shape=jax.ShapeDtypeStruct((NC * N, H), jnp.float32),
    )(x, W1, deg_out)


def _mm2_body(a0_ref, a1_ref, w_ref, b_ref, dego_ref, degi_ref, o_ref):
    nd = _norm(degi_ref[...])
    ns = _norm(dego_ref[...])
    h = jnp.concatenate([a0_ref[...], a1_ref[...]], axis=1)
    h = jnp.maximum(h * nd + b_ref[...], 0.0)
    t = lax.dot_general(h, w_ref[...], (((1,), (1,)), ((), ())),
                        preferred_element_type=jnp.float32)
    o_ref[...] = t * ns


def _mm2(agg1, W2, b1, deg_out, deg_in):
    return pl.pallas_call(
        _mm2_body,
        grid=(NB, NC),
        in_specs=[
            pl.BlockSpec((BN, H), lambda i, c: (i, 0)),
            pl.BlockSpec((BN, H), lambda i, c: (NB + i, 0)),
            pl.BlockSpec((H, D), lambda i, c: (c, 0)),
            pl.BlockSpec((1, D), lambda i, c: (0, 0)),
            pl.BlockSpec((BN, 1), lambda i, c: (i, 0)),
            pl.BlockSpec((BN, 1), lambda i, c: (i, 0)),
        ],
        out_specs=pl.BlockSpec((BN, H), lambda i, c: (c * NB + i, 0)),
        out_shape=jax.ShapeDtypeStruct((NC * N, H), jnp.float32),
    )(agg1, agg1, W2, b1, deg_out, deg_in)


def _sn_row(w):
    return w / jnp.maximum(jnp.sqrt(jnp.sum(w * w)), 1e-12)


def _heads_body(a0_ref, a1_ref, degi_ref, b2_ref,
                wp0_ref, bp0_ref, wp1_ref, bp1_ref, wv_ref, bv_ref,
                o_ref, v_ref):
    i = pl.program_id(0)
    nd = _norm(degi_ref[...])
    h2 = jnp.concatenate([a0_ref[...], a1_ref[...]], axis=1)
    h2 = h2 * nd + b2_ref[...]
    U = jnp.concatenate(
        [_sn_row(wp0_ref[...]), _sn_row(wp1_ref[...]), _sn_row(wv_ref[...]),
         jnp.zeros((H - 3, D), jnp.float32)], axis=0)
    P = lax.dot_general(h2, U, (((1,), (1,)), ((), ())),
                        preferred_element_type=jnp.float32)
    lane = lax.broadcasted_iota(jnp.int32, (BN, H), 1)
    bvec = jnp.where(lane == 0, bp0_ref[0],
                     jnp.where(lane == 1, bp1_ref[0], 0.0))
    o_ref[...] = P + bvec
    pv = jnp.sum(jnp.where(lane == 2, P, 0.0))

    @pl.when(i == 0)
    def _():
        v_ref[0] = 0.0
    v_ref[0] = v_ref[0] + pv

    @pl.when(i == NB - 1)
    def _():
        v_ref[0] = v_ref[0] / N + bv_ref[0]


def _heads(agg2, deg_in, b2, w_pi0, b_pi0, w_pi1, b_pi1, w_v, b_v):
    full = lambda i: (0, 0)
    return pl.pallas_call(
        _heads_body,
        grid=(NB,),
        in_specs=[
            pl.BlockSpec((BN, H), lambda i: (i, 0)),
            pl.BlockSpec((BN, H), lambda i: (NB + i, 0)),
            pl.BlockSpec((BN, 1), lambda i: (i, 0)),
            pl.BlockSpec((1, D), full),
            pl.BlockSpec((1, D), full),
            pl.BlockSpec(memory_space=pltpu.MemorySpace.SMEM),
            pl.BlockSpec((1, D), full),
            pl.BlockSpec(memory_space=pltpu.MemorySpace.SMEM),
            pl.BlockSpec((1, D), full),
            pl.BlockSpec(memory_space=pltpu.MemorySpace.SMEM),
        ],
        out_specs=[
            pl.BlockSpec((BN, H), lambda i: (i, 0)),
            pl.BlockSpec(memory_space=pltpu.MemorySpace.SMEM),
        ],
        out_shape=[
            jax.ShapeDtypeStruct((N, H), jnp.float32),
            jax.ShapeDtypeStruct((1,), jnp.float32),
        ],
    )(agg2, agg2, deg_in, b2, w_pi0, b_pi0, w_pi1, b_pi1, w_v, b_v)


def kernel(x, edge_index, edge_weight, W1, b1, W2, b2,
           w_pi0, b_pi0, w_pi1, b_pi1, w_v, b_v):
    pad = R * EB - E
    fill = jnp.arange(pad, dtype=jnp.int32) % N
    src2 = jnp.concatenate([edge_index[0], fill]).reshape(R, EB)
    dst2 = jnp.concatenate([edge_index[1], fill]).reshape(R, EB)
    ew1 = jnp.concatenate([edge_weight, jnp.zeros((pad,), jnp.float32)])
    ew2 = ew1.reshape(R, EB)

    deg = _sc_degrees(src2, dst2, ew2)
    deg_out = deg[:N].reshape(N, 1)
    deg_in = deg[NDEG:NDEG + N].reshape(N, 1)

    t1 = _mm1(x, W1, deg_out)
    agg1 = _sc_propagate(t1, src2, dst2, ew1)
    t2 = _mm2(agg1, W2, b1.reshape(1, D), deg_out, deg_in)
    agg2 = _sc_propagate(t2, src2, dst2, ew1)
    ph, v = _heads(agg2, deg_in, b2.reshape(1, D),
                   w_pi0, b_pi0, w_pi1, b_pi1, w_v, b_v)
    pi = jnp.concatenate([ph[:, 0:1], ph[:, 1:2]], axis=0)
    return (pi, v.reshape(1, 1))


# speedup vs baseline: 7.3008x; 1.0266x over previous
import functools

import jax
import jax.numpy as jnp
from jax import lax
from jax.experimental import pallas as pl
from jax.experimental.pallas import tpu as pltpu
from jax.experimental.pallas import tpu_sc as plsc

N = 10000
E = 160000
D = 256
H = 128
NC = 2
NS = 16
EB = 128
R = 1280
RPT = R // NS
MC = RPT // 8
NPT = 624
BN = 400
NB = N // BN
DEGP = 640
NDEG = NS * DEGP

_mesh = plsc.VectorSubcoreMesh(
    core_axis_name="c", subcore_axis_name="s", num_cores=NC, num_subcores=NS)


def _deg_body(idx_ref, ew_ref, out_ref, acc,
              ibufa, ibufb, wbufa, wbufb, zbuf, sia, sib, swa, swb):
    c = lax.axis_index("c")
    s = lax.axis_index("s")

    def zloop(i, _):
        zbuf[pl.ds(i * 16, 16)] = jnp.zeros((16,), jnp.float32)
        return 0
    lax.fori_loop(0, DEGP // 16, zloop, 0)
    pltpu.sync_copy(zbuf, acc.at[pl.ds(s * DEGP, DEGP)])

    ibufs = (ibufa, ibufb)
    wbufs = (wbufa, wbufb)
    isems = (sia, sib)
    wsems = (swa, swb)
    r0 = s * RPT
    di = {}
    dw = {}
    dsc = {0: [], 1: []}
    di[0] = pltpu.async_copy(idx_ref.at[c].at[pl.ds(r0, 8)], ibufa, sia)
    dw[0] = pltpu.async_copy(ew_ref.at[pl.ds(r0, 8)], wbufa, swa)
    plsc.subcore_barrier()
    for m in range(MC):
        p = m & 1
        q = (m + 1) & 1
        di[m].wait()
        dw[m].wait()
        if m + 1 < MC:
            for d in dsc[q]:
                d.wait()
            dsc[q] = []
            base = r0 + (m + 1) * 8
            di[m + 1] = pltpu.async_copy(
                idx_ref.at[c].at[pl.ds(base, 8)], ibufs[q], isems[q])
            dw[m + 1] = pltpu.async_copy(
                ew_ref.at[pl.ds(base, 8)], wbufs[q], wsems[q])
        for j in range(8):
            dsc[p].append(pltpu.async_copy(
                wbufs[p].at[j], acc.at[ibufs[p].at[j]], wsems[p], add=True))
    for d in dsc[0]:
        d.wait()
    for d in dsc[1]:
        d.wait()
    plsc.subcore_barrier()
    pltpu.sync_copy(acc.at[pl.ds(s * DEGP, DEGP)],
                    out_ref.at[pl.ds((c * NS + s) * DEGP, DEGP)])


@functools.partial(
    pl.kernel,
    out_type=jax.ShapeDtypeStruct((NC * NDEG,), jnp.float32),
    mesh=_mesh,
    scratch_types=[
        pltpu.MemorySpace.VMEM_SHARED((NDEG,), jnp.float32),
        pltpu.VMEM((8, EB), jnp.int32),
        pltpu.VMEM((8, EB), jnp.int32),
        pltpu.VMEM((8, EB), jnp.float32),
        pltpu.VMEM((8, EB), jnp.float32),
        pltpu.VMEM((DEGP,), jnp.float32),
        pltpu.SemaphoreType.DMA,
        pltpu.SemaphoreType.DMA,
        pltpu.SemaphoreType.DMA,
        pltpu.SemaphoreType.DMA,
    ],
)
def _sc_degrees(idx_ref, ew_ref, out_ref, acc,
                ibufa, ibufb, wbufa, wbufb, zbuf, sia, sib, swa, swb):
    _deg_body(idx_ref, ew_ref, out_ref, acc,
              ibufa, ibufb, wbufa, wbufb, zbuf, sia, sib, swa, swb)


def _prop_body(t_ref, src_ref, dst_ref, ew_ref, out_ref,
               acc, sbuf, dbuf, wbuf, rowsa, rowsb, gsa, gsb, ssa, ssb):
    c = lax.axis_index("c")
    s = lax.axis_index("s")
    c_off = c * N

    def zloop(i, _):
        for k in range(H // 16):
            rowsa[i, pl.ds(k * 16, 16)] = jnp.zeros((16,), jnp.float32)
        return 0
    lax.fori_loop(0, EB, zloop, 0)
    for q in range(4):
        pltpu.sync_copy(rowsa, acc.at[pl.ds(s * NPT + q * EB, EB)])

    @pl.when(s < NS - 1)
    def _():
        pltpu.sync_copy(rowsa.at[pl.ds(0, NPT - 4 * EB)],
                        acc.at[pl.ds(s * NPT + 4 * EB, NPT - 4 * EB)])

    @pl.when(s == NS - 1)
    def _():
        pltpu.sync_copy(rowsa, acc.at[pl.ds((NS - 1) * NPT + 4 * EB, EB)])
    plsc.subcore_barrier()

    bufs = (rowsa, rowsb)
    gsems = (gsa, gsb)
    ssems = (ssa, ssb)

    def chunk_body(m, _):
        base = s * RPT + m * 8
        pltpu.sync_copy(src_ref.at[pl.ds(base, 8)], sbuf)
        pltpu.sync_copy(dst_ref.at[pl.ds(base, 8)], dbuf)
        pltpu.sync_copy(ew_ref.at[pl.ds(base * EB, 8 * EB)], wbuf)

        @pl.when(c == 1)
        def _():
            for j in range(8):
                for g in range(EB // 16):
                    sbuf[j, pl.ds(g * 16, 16)] = sbuf[j, pl.ds(g * 16, 16)] + c_off
        dg = {0: pltpu.async_copy(t_ref.at[sbuf.at[0]], rowsa, gsa)}
        dpend = {}
        for j in range(8):
            p = j & 1
            dg[j].wait()
            if j < 7:
                q = (j + 1) & 1
                if j >= 1:
                    dpend[q].wait()
                dg[j + 1] = pltpu.async_copy(
                    t_ref.at[sbuf.at[j + 1]], bufs[q], gsems[q])
            buf = bufs[p]

            def mul_body(g, _):
                cvec = wbuf[pl.ds(j * EB + g * 16, 16)]
                for l in range(16):
                    e = g * 16 + l
                    coef = cvec[l]
                    for k in range(H // 16):
                        buf[e, pl.ds(k * 16, 16)] = buf[e, pl.ds(k * 16, 16)] * coef
                return 0
            lax.fori_loop(0, EB // 16, mul_body, 0, unroll=2)
            dpend[p] = pltpu.async_copy(buf, acc.at[dbuf.at[j]],
                                        ssems[p], add=True)
        dpend[0].wait()
        dpend[1].wait()
        return 0
    lax.fori_loop(0, MC, chunk_body, 0)
    plsc.subcore_barrier()

    @pl.when(s < NS - 1)
    def _():
        pltpu.sync_copy(acc.at[pl.ds(s * NPT, NPT)],
                        out_ref.at[pl.ds(c_off + s * NPT, NPT)])

    @pl.when(s == NS - 1)
    def _():
        pltpu.sync_copy(acc.at[pl.ds((NS - 1) * NPT, 640)],
                        out_ref.at[pl.ds(c_off + (NS - 1) * NPT, 640)])


@functools.partial(
    pl.kernel,
    out_type=jax.ShapeDtypeStruct((NC * N, H), jnp.float32),
    mesh=_mesh,
    scratch_types=[
        pltpu.MemorySpace.VMEM_SHARED((N, H), jnp.float32),
        pltpu.VMEM((8, EB), jnp.int32),
        pltpu.VMEM((8, EB), jnp.int32),
        pltpu.VMEM((8 * EB,), jnp.float32),
        pltpu.VMEM((EB, H), jnp.float32),
        pltpu.VMEM((EB, H), jnp.float32),
        pltpu.SemaphoreType.DMA,
        pltpu.SemaphoreType.DMA,
        pltpu.SemaphoreType.DMA,
        pltpu.SemaphoreType.DMA,
    ],
)
def _sc_propagate(t_ref, src_ref, dst_ref, ew_ref, out_ref,
                  acc, sbuf, dbuf, wbuf, rowsa, rowsb, gsa, gsb, ssa, ssb):
    _prop_body(t_ref, src_ref, dst_ref, ew_ref, out_ref,
               acc, sbuf, dbuf, wbuf, rowsa, rowsb, gsa, gsb, ssa, ssb)


def _norm(deg_slice):
    return lax.rsqrt(jnp.maximum(deg_slice, 1.0))


def _mm1_body(x_ref, w_ref, dego_ref, o_ref):
    ns = _norm(dego_ref[...])
    t = lax.dot_general(x_ref[...], w_ref[...], (((1,), (1,)), ((), ())),
                        preferred_element_type=jnp.float32)
    o_ref[...] = t * ns


def _mm1(x, W1, deg_out):
    return pl.pallas_call(
        _mm1_body,
        grid=(NB, NC),
        in_specs=[
            pl.BlockSpec((BN, D), lambda i, c: (i, 0)),
            pl.BlockSpec((H, D), lambda i, c: (c, 0)),
            pl.BlockSpec((BN, 1), lambda i, c: (i, 0)),
        ],
        out_specs=pl.BlockSpec((BN, H), lambda i, c: (c * NB + i, 0)),
        out_shape=jax.ShapeDtypeStruct((NC * N, H), jnp.float32),
    )(x, W1, deg_out)


def _mm2_body(a0_ref, a1_ref, w_ref, b_ref, dego_ref, degi_ref, o_ref):
    nd = _norm(degi_ref[...])
    ns = _norm(dego_ref[...])
    h = jnp.concatenate([a0_ref[...], a1_ref[...]], axis=1)
    h = jnp.maximum(h * nd + b_ref[...], 0.0)
    t = lax.dot_general(h, w_ref[...], (((1,), (1,)), ((), ())),
                        preferred_element_type=jnp.float32)
    o_ref[...] = t * ns


def _mm2(agg1, W2, b1, deg_out, deg_in):
    return pl.pallas_call(
        _mm2_body,
        grid=(NB, NC),
        in_specs=[
            pl.BlockSpec((BN, H), lambda i, c: (i, 0)),
            pl.BlockSpec((BN, H), lambda i, c: (NB + i, 0)),
            pl.BlockSpec((H, D), lambda i, c: (c, 0)),
            pl.BlockSpec((1, D), lambda i, c: (0, 0)),
            pl.BlockSpec((BN, 1), lambda i, c: (i, 0)),
            pl.BlockSpec((BN, 1), lambda i, c: (i, 0)),
        ],
        out_specs=pl.BlockSpec((BN, H), lambda i, c: (c * NB + i, 0)),
        out_shape=jax.ShapeDtypeStruct((NC * N, H), jnp.float32),
    )(agg1, agg1, W2, b1, deg_out, deg_in)


def _sn_row(w):
    return w / jnp.maximum(jnp.sqrt(jnp.sum(w * w)), 1e-12)


def _heads_body(a0_ref, a1_ref, degi_ref, b2_ref,
                wp0_ref, bp0_ref, wp1_ref, bp1_ref, wv_ref, bv_ref,
                o_ref, v_ref):
    i = pl.program_id(0)
    nd = _norm(degi_ref[...])
    h2 = jnp.concatenate([a0_ref[...], a1_ref[...]], axis=1)
    h2 = h2 * nd + b2_ref[...]
    U = jnp.concatenate(
        [_sn_row(wp0_ref[...]), _sn_row(wp1_ref[...]), _sn_row(wv_ref[...]),
         jnp.zeros((H - 3, D), jnp.float32)], axis=0)
    P = lax.dot_general(h2, U, (((1,), (1,)), ((), ())),
                        preferred_element_type=jnp.float32)
    lane = lax.broadcasted_iota(jnp.int32, (BN, H), 1)
    bvec = jnp.where(lane == 0, bp0_ref[0],
                     jnp.where(lane == 1, bp1_ref[0], 0.0))
    o_ref[...] = P + bvec
    pv = jnp.sum(jnp.where(lane == 2, P, 0.0))

    @pl.when(i == 0)
    def _():
        v_ref[0] = 0.0
    v_ref[0] = v_ref[0] + pv

    @pl.when(i == NB - 1)
    def _():
        v_ref[0] = v_ref[0] / N + bv_ref[0]


def _heads(agg2, deg_in, b2, w_pi0, b_pi0, w_pi1, b_pi1, w_v, b_v):
    full = lambda i: (0, 0)
    return pl.pallas_call(
        _heads_body,
        grid=(NB,),
        in_specs=[
            pl.BlockSpec((BN, H), lambda i: (i, 0)),
            pl.BlockSpec((BN, H), lambda i: (NB + i, 0)),
            pl.BlockSpec((BN, 1), lambda i: (i, 0)),
            pl.BlockSpec((1, D), full),
            pl.BlockSpec((1, D), full),
            pl.BlockSpec(memory_space=pltpu.MemorySpace.SMEM),
            pl.BlockSpec((1, D), full),
            pl.BlockSpec(memory_space=pltpu.MemorySpace.SMEM),
            pl.BlockSpec((1, D), full),
            pl.BlockSpec(memory_space=pltpu.MemorySpace.SMEM),
        ],
        out_specs=[
            pl.BlockSpec((BN, H), lambda i: (i, 0)),
            pl.BlockSpec(memory_space=pltpu.MemorySpace.SMEM),
        ],
        out_shape=[
            jax.ShapeDtypeStruct((N, H), jnp.float32),
            jax.ShapeDtypeStruct((1,), jnp.float32),
        ],
    )(agg2, agg2, deg_in, b2, w_pi0, b_pi0, w_pi1, b_pi1, w_v, b_v)


def kernel(x, edge_index, edge_weight, W1, b1, W2, b2,
           w_pi0, b_pi0, w_pi1, b_pi1, w_v, b_v):
    pad = R * EB - E
    fill = jnp.arange(pad, dtype=jnp.int32) % N
    src2 = jnp.concatenate([edge_index[0], fill]).reshape(R, EB)
    dst2 = jnp.concatenate([edge_index[1], fill]).reshape(R, EB)
    ew1 = jnp.concatenate([edge_weight, jnp.zeros((pad,), jnp.float32)])
    ew2 = ew1.reshape(R, EB)

    idx2 = jnp.stack([src2, dst2])
    deg = _sc_degrees(idx2, ew2)
    deg_out = deg[:N].reshape(N, 1)
    deg_in = deg[NDEG:NDEG + N].reshape(N, 1)

    t1 = _mm1(x, W1, deg_out)
    agg1 = _sc_propagate(t1, src2, dst2, ew1)
    t2 = _mm2(agg1, W2, b1.reshape(1, D), deg_out, deg_in)
    agg2 = _sc_propagate(t2, src2, dst2, ew1)
    ph, v = _heads(agg2, deg_in, b2.reshape(1, D),
                   w_pi0, b_pi0, w_pi1, b_pi1, w_v, b_v)
    pi = jnp.concatenate([ph[:, 0:1], ph[:, 1:2]], axis=0)
    return (pi, v.reshape(1, 1))
